# baseline (speedup 1.0000x reference)
import numpy as np
import jax
import jax.numpy as jnp
from jax.experimental import pallas as pl
from jax.experimental.pallas import tpu as pltpu

_BN = 512
_G1 = 6 * 14 * 14
_G2 = 16 * 5 * 5


def _band(src, half, par):
    a = np.zeros((src, half, 5), np.float32)
    for h in range(half):
        for d in range(5):
            a[2 * h + par + d, h, d] = 1.0
    return a


_A1 = (_band(32, 14, 0), _band(32, 14, 1))
_A2 = (_band(14, 5, 0), _band(14, 5, 1))

_A1G = np.stack([_A1[py] for py in (0, 1) for px in (0, 1)])
_B1G = np.stack([_A1[px] for py in (0, 1) for px in (0, 1)])
_A2G = np.stack([_A2[py] for py in (0, 1) for px in (0, 1)])
_B2G = np.stack([_A2[px] for py in (0, 1) for px in (0, 1)])

_P2PERM = np.arange(400).reshape(16, 5, 5).transpose(1, 2, 0).reshape(400)


def _lenet_block(x_ref, m1_ref, b1_ref, m2_ref, b2_ref,
                 w3_ref, b3_ref, w4_ref, b4_ref, w5_ref, b5_ref, o_ref):
    f32 = jnp.float32
    bf16 = jnp.bfloat16

    xb = x_ref[...].astype(bf16)

    y1 = jnp.dot(xb, m1_ref[...], preferred_element_type=f32)
    y1 = jnp.maximum(y1 + b1_ref[...], 0.0)

    p1 = jnp.maximum(
        jnp.maximum(y1[:, 0:_G1], y1[:, _G1:2 * _G1]),
        jnp.maximum(y1[:, 2 * _G1:3 * _G1], y1[:, 3 * _G1:4 * _G1]))

    y2 = jnp.dot(p1.astype(bf16), m2_ref[...],
                 preferred_element_type=f32)
    y2 = jnp.maximum(y2 + b2_ref[...], 0.0)

    p2 = jnp.maximum(
        jnp.maximum(y2[:, 0:_G2], y2[:, _G2:2 * _G2]),
        jnp.maximum(y2[:, 2 * _G2:3 * _G2], y2[:, 3 * _G2:4 * _G2]))

    h1 = jnp.maximum(jnp.dot(p2.astype(bf16), w3_ref[...],
                             preferred_element_type=f32) + b3_ref[...], 0.0)
    h2 = jnp.maximum(jnp.dot(h1.astype(bf16), w4_ref[...],
                             preferred_element_type=f32) + b4_ref[...], 0.0)
    o_ref[...] = jnp.dot(h2.astype(bf16), w5_ref[...],
                         preferred_element_type=f32) + b5_ref[...]


@jax.jit
def kernel(x, conv1_w, conv1_b, conv2_w, conv2_b,
           fc1_w, fc1_b, fc2_w, fc2_b, fc3_w, fc3_b):
    bf16 = jnp.bfloat16
    B = x.shape[0]
    x2d = x.reshape(B, 32 * 32)

    w1b = conv1_w.reshape(6, 5, 5).astype(bf16)
    a1g = jnp.asarray(_A1G, bf16)
    b1g = jnp.asarray(_B1G, bf16)
    m1 = jnp.einsum('kde,gyhd,gxwe->yxghwk', w1b, a1g, b1g,
                    ).reshape(1024, 4 * _G1)
    b1 = jnp.broadcast_to(conv1_b[None, :], (784, 6)).reshape(1, 4 * _G1)

    w2b = conv2_w.astype(bf16)
    a2g = jnp.asarray(_A2G, bf16)
    b2g = jnp.asarray(_B2G, bf16)
    m2 = jnp.einsum('kcde,gyhd,gxwe->yxcghwk', w2b, a2g, b2g,
                    ).reshape(_G1, 4 * _G2)
    b2 = jnp.broadcast_to(conv2_b[None, :], (100, 16)).reshape(1, 4 * _G2)

    w3 = fc1_w[:, _P2PERM].T.astype(bf16)
    w4 = fc2_w.T.astype(bf16)
    w5 = fc3_w.T.astype(bf16)
    b3 = fc1_b.reshape(1, 120)
    b4 = fc2_b.reshape(1, 84)
    b5 = fc3_b.reshape(1, 10)

    pad = (-B) % _BN
    if pad:
        x2d = jnp.pad(x2d, ((0, pad), (0, 0)))
    bp = B + pad

    def const(a):
        return pl.BlockSpec(a.shape, lambda i, _nd=a.ndim: (0,) * _nd)

    out = pl.pallas_call(
        _lenet_block,
        out_shape=jax.ShapeDtypeStruct((bp, 10), jnp.float32),
        grid=(bp // _BN,),
        in_specs=[
            pl.BlockSpec((_BN, 1024), lambda i: (i, 0)),
            const(m1), const(b1), const(m2), const(b2),
            const(w3), const(b3), const(w4), const(b4), const(w5), const(b5),
        ],
        out_specs=pl.BlockSpec((_BN, 10), lambda i: (i, 0)),
        compiler_params=pltpu.CompilerParams(
            dimension_semantics=("arbitrary",)),
    )(x2d, m1, b1, m2, b2, w3, b3, w4, b4, w5, b5)
    return out[:B] if pad else out


# speedup vs baseline: 1.7784x; 1.7784x over previous
import numpy as np
import jax
import jax.numpy as jnp
from jax.experimental import pallas as pl
from jax.experimental.pallas import tpu as pltpu

_BN = 512
_G1 = 6 * 14 * 14
_G2 = 16 * 5 * 5


def _band(src, half, par):
    a = np.zeros((src, half, 5), np.float32)
    for h in range(half):
        for d in range(5):
            a[2 * h + par + d, h, d] = 1.0
    return a


_A1 = (_band(32, 14, 0), _band(32, 14, 1))
_A2 = (_band(14, 5, 0), _band(14, 5, 1))

_A1G = np.stack([_A1[py] for py in (0, 1) for px in (0, 1)])
_B1G = np.stack([_A1[px] for py in (0, 1) for px in (0, 1)])
_A2G = np.stack([_A2[py] for py in (0, 1) for px in (0, 1)])
_B2G = np.stack([_A2[px] for py in (0, 1) for px in (0, 1)])

_P2PERM = np.arange(400).reshape(16, 5, 5).transpose(1, 2, 0).reshape(400)


def _lenet_block(x_ref, m1_ref, b1_ref, m2_ref, b2_ref,
                 w3_ref, b3_ref, w4_ref, b4_ref, w5_ref, b5_ref, o_ref):
    f32 = jnp.float32
    bf16 = jnp.bfloat16

    xb = x_ref[...].astype(bf16)

    y1 = jnp.dot(xb, m1_ref[...], preferred_element_type=f32)
    y1 = jnp.maximum(y1 + b1_ref[...], 0.0)

    p1 = jnp.maximum(
        jnp.maximum(y1[:, 0:_G1], y1[:, _G1:2 * _G1]),
        jnp.maximum(y1[:, 2 * _G1:3 * _G1], y1[:, 3 * _G1:4 * _G1]))

    y2 = jnp.dot(p1.astype(bf16), m2_ref[...],
                 preferred_element_type=f32)
    y2 = jnp.maximum(y2 + b2_ref[...], 0.0)

    p2 = jnp.maximum(
        jnp.maximum(y2[:, 0:_G2], y2[:, _G2:2 * _G2]),
        jnp.maximum(y2[:, 2 * _G2:3 * _G2], y2[:, 3 * _G2:4 * _G2]))

    h1 = jnp.maximum(jnp.dot(p2.astype(bf16), w3_ref[...],
                             preferred_element_type=f32) + b3_ref[...], 0.0)
    h2 = jnp.maximum(jnp.dot(h1.astype(bf16), w4_ref[...],
                             preferred_element_type=f32) + b4_ref[...], 0.0)
    o_ref[...] = jnp.dot(h2.astype(bf16), w5_ref[...],
                         preferred_element_type=f32) + b5_ref[...]


@jax.jit
def kernel(x, conv1_w, conv1_b, conv2_w, conv2_b,
           fc1_w, fc1_b, fc2_w, fc2_b, fc3_w, fc3_b):
    bf16 = jnp.bfloat16
    B = x.shape[0]
    x2d = x.reshape(B, 32 * 32)

    w1b = conv1_w.reshape(6, 5, 5).astype(bf16)
    s1 = [jnp.einsum('kde,xwe->dxwk', w1b, jnp.asarray(_A1[px], bf16)
                     ).reshape(160, 84) for px in (0, 1)]
    m1 = jnp.concatenate(
        [jnp.pad(s1[px], ((64 * h + 32 * py, 864 - 64 * h - 32 * py), (0, 0)))
         for py in (0, 1) for px in (0, 1) for h in range(14)],
        axis=1)
    b1 = jnp.broadcast_to(conv1_b[None, :], (784, 6)).reshape(1, 4 * _G1)

    w2b = conv2_w.astype(bf16)
    s2 = [jnp.einsum('kcde,xwe->dxcwk', w2b, jnp.asarray(_A2[px], bf16)
                     ).reshape(420, 80) for px in (0, 1)]
    m2 = jnp.concatenate(
        [jnp.pad(s2[px], (((2 * h + py) * 84, 756 - (2 * h + py) * 84), (0, 0)))
         for py in (0, 1) for px in (0, 1) for h in range(5)],
        axis=1)
    b2 = jnp.broadcast_to(conv2_b[None, :], (100, 16)).reshape(1, 4 * _G2)

    w3 = fc1_w[:, _P2PERM].T.astype(bf16)
    w4 = fc2_w.T.astype(bf16)
    w5 = fc3_w.T.astype(bf16)
    b3 = fc1_b.reshape(1, 120)
    b4 = fc2_b.reshape(1, 84)
    b5 = fc3_b.reshape(1, 10)

    pad = (-B) % _BN
    if pad:
        x2d = jnp.pad(x2d, ((0, pad), (0, 0)))
    bp = B + pad

    def const(a):
        return pl.BlockSpec(a.shape, lambda i, _nd=a.ndim: (0,) * _nd)

    out = pl.pallas_call(
        _lenet_block,
        out_shape=jax.ShapeDtypeStruct((bp, 10), jnp.float32),
        grid=(bp // _BN,),
        in_specs=[
            pl.BlockSpec((_BN, 1024), lambda i: (i, 0)),
            const(m1), const(b1), const(m2), const(b2),
            const(w3), const(b3), const(w4), const(b4), const(w5), const(b5),
        ],
        out_specs=pl.BlockSpec((_BN, 10), lambda i: (i, 0)),
        compiler_params=pltpu.CompilerParams(
            dimension_semantics=("arbitrary",)),
    )(x2d, m1, b1, m2, b2, w3, b3, w4, b4, w5, b5)
    return out[:B] if pad else out


# speedup vs baseline: 1.8633x; 1.0478x over previous
import numpy as np
import jax
import jax.numpy as jnp
from jax.experimental import pallas as pl
from jax.experimental.pallas import tpu as pltpu

_BN = 512
_S1 = 1280
_S2 = 512
_G1 = 6 * 14 * 14
_G2 = 16 * 5 * 5


def _band(src, half, par):
    a = np.zeros((src, half, 5), np.float32)
    for w in range(half):
        for e in range(5):
            a[2 * w + par + e, w, e] = 1.0
    return a


_A1 = (_band(32, 14, 0), _band(32, 14, 1))
_A2 = (_band(14, 5, 0), _band(14, 5, 1))

_P2PERM = np.arange(400).reshape(16, 5, 5).transpose(1, 2, 0).reshape(400)


def _lenet_block(x_ref, m1_ref, b1_ref, m2_ref, b2_ref,
                 w3_ref, b3_ref, w4_ref, b4_ref, w5_ref, b5_ref, o_ref):
    f32 = jnp.float32
    bf16 = jnp.bfloat16

    xb = x_ref[...].astype(bf16)

    y1 = jnp.dot(xb, m1_ref[...], preferred_element_type=f32)

    p1 = jnp.maximum(
        jnp.maximum(y1[:, 0:_S1], y1[:, _S1:2 * _S1]),
        jnp.maximum(y1[:, 2 * _S1:3 * _S1], y1[:, 3 * _S1:4 * _S1]))
    p1 = jnp.maximum(p1 + b1_ref[...], 0.0)

    y2 = jnp.dot(p1.astype(bf16), m2_ref[...],
                 preferred_element_type=f32)

    p2 = jnp.maximum(
        jnp.maximum(y2[:, 0:_S2], y2[:, _S2:2 * _S2]),
        jnp.maximum(y2[:, 2 * _S2:3 * _S2], y2[:, 3 * _S2:4 * _S2]))
    p2 = jnp.maximum(p2 + b2_ref[...], 0.0)

    h1 = jnp.maximum(jnp.dot(p2.astype(bf16), w3_ref[...],
                             preferred_element_type=f32) + b3_ref[...], 0.0)
    h2 = jnp.maximum(jnp.dot(h1.astype(bf16), w4_ref[...],
                             preferred_element_type=f32) + b4_ref[...], 0.0)
    o_ref[...] = jnp.dot(h2.astype(bf16), w5_ref[...],
                         preferred_element_type=f32) + b5_ref[...]


@jax.jit
def kernel(x, conv1_w, conv1_b, conv2_w, conv2_b,
           fc1_w, fc1_b, fc2_w, fc2_b, fc3_w, fc3_b):
    bf16 = jnp.bfloat16
    B = x.shape[0]
    x2d = x.reshape(B, 32 * 32)

    w1b = conv1_w.reshape(6, 5, 5).astype(bf16)
    s1 = [jnp.einsum('kde,xwe->dxwk', w1b, jnp.asarray(_A1[px], bf16)
                     ).reshape(160, 84) for px in (0, 1)]
    z1 = jnp.zeros((1024, _S1 - _G1), bf16)
    m1 = jnp.concatenate(
        [blk for py in (0, 1) for px in (0, 1) for blk in
         [jnp.pad(s1[px], ((64 * h + 32 * py, 864 - 64 * h - 32 * py), (0, 0)))
          for h in range(14)] + [z1]],
        axis=1)
    b1 = jnp.pad(
        jnp.broadcast_to(conv1_b[None, :], (196, 6)).reshape(1, _G1),
        ((0, 0), (0, _S1 - _G1)))

    w2b = conv2_w.astype(bf16)
    s2 = [jnp.einsum('kcde,xwe->dxcwk', w2b, jnp.asarray(_A2[px], bf16)
                     ).reshape(420, 80) for px in (0, 1)]
    z2 = jnp.zeros((_S1, _S2 - _G2), bf16)
    m2 = jnp.concatenate(
        [blk for py in (0, 1) for px in (0, 1) for blk in
         [jnp.pad(s2[px], (((2 * h + py) * 84,
                            _S1 - 420 - (2 * h + py) * 84), (0, 0)))
          for h in range(5)] + [z2]],
        axis=1)
    b2 = jnp.pad(
        jnp.broadcast_to(conv2_b[None, :], (25, 16)).reshape(1, _G2),
        ((0, 0), (0, _S2 - _G2)))

    w3 = jnp.pad(fc1_w[:, _P2PERM].T.astype(bf16),
                 ((0, _S2 - _G2), (0, 0)))
    w4 = fc2_w.T.astype(bf16)
    w5 = fc3_w.T.astype(bf16)
    b3 = fc1_b.reshape(1, 120)
    b4 = fc2_b.reshape(1, 84)
    b5 = fc3_b.reshape(1, 10)

    pad = (-B) % _BN
    if pad:
        x2d = jnp.pad(x2d, ((0, pad), (0, 0)))
    bp = B + pad

    def const(a):
        return pl.BlockSpec(a.shape, lambda i, _nd=a.ndim: (0,) * _nd)

    out = pl.pallas_call(
        _lenet_block,
        out_shape=jax.ShapeDtypeStruct((bp, 10), jnp.float32),
        grid=(bp // _BN,),
        in_specs=[
            pl.BlockSpec((_BN, 1024), lambda i: (i, 0)),
            const(m1), const(b1), const(m2), const(b2),
            const(w3), const(b3), const(w4), const(b4), const(w5), const(b5),
        ],
        out_specs=pl.BlockSpec((_BN, 10), lambda i: (i, 0)),
        compiler_params=pltpu.CompilerParams(
            dimension_semantics=("parallel",)),
    )(x2d, m1, b1, m2, b2, w3, b3, w4, b4, w5, b5)
    return out[:B] if pad else out


# speedup vs baseline: 1.9027x; 1.0211x over previous
import numpy as np
import jax
import jax.numpy as jnp
from jax.experimental import pallas as pl
from jax.experimental.pallas import tpu as pltpu

_BN = 1024
_S1 = 1280
_S2 = 512
_G1 = 6 * 14 * 14
_G2 = 16 * 5 * 5


def _band(src, half, par):
    a = np.zeros((src, half, 5), np.float32)
    for w in range(half):
        for e in range(5):
            a[2 * w + par + e, w, e] = 1.0
    return a


_A1 = (_band(32, 14, 0), _band(32, 14, 1))
_A2 = (_band(14, 5, 0), _band(14, 5, 1))

_P2PERM = np.arange(400).reshape(16, 5, 5).transpose(1, 2, 0).reshape(400)


def _lenet_block(x_ref, m1_ref, b1_ref, m2_ref, b2_ref,
                 w3_ref, b3_ref, w4_ref, b4_ref, w5_ref, b5_ref, o_ref):
    f32 = jnp.float32
    bf16 = jnp.bfloat16

    xb = x_ref[...].astype(bf16)

    p1 = jnp.dot(xb, m1_ref[:, 0:_S1], preferred_element_type=f32)
    for g in range(1, 4):
        p1 = jnp.maximum(p1, jnp.dot(xb, m1_ref[:, g * _S1:(g + 1) * _S1],
                                     preferred_element_type=f32))
    p1 = jnp.maximum(p1 + b1_ref[...], 0.0)

    p1b = p1.astype(bf16)
    p2 = jnp.dot(p1b, m2_ref[:, 0:_S2], preferred_element_type=f32)
    for g in range(1, 4):
        p2 = jnp.maximum(p2, jnp.dot(p1b, m2_ref[:, g * _S2:(g + 1) * _S2],
                                     preferred_element_type=f32))
    p2 = jnp.maximum(p2 + b2_ref[...], 0.0)

    h1 = jnp.maximum(jnp.dot(p2.astype(bf16), w3_ref[...],
                             preferred_element_type=f32) + b3_ref[...], 0.0)
    h2 = jnp.maximum(jnp.dot(h1.astype(bf16), w4_ref[...],
                             preferred_element_type=f32) + b4_ref[...], 0.0)
    o_ref[...] = jnp.dot(h2.astype(bf16), w5_ref[...],
                         preferred_element_type=f32) + b5_ref[...]


@jax.jit
def kernel(x, conv1_w, conv1_b, conv2_w, conv2_b,
           fc1_w, fc1_b, fc2_w, fc2_b, fc3_w, fc3_b):
    bf16 = jnp.bfloat16
    B = x.shape[0]
    x2d = x.reshape(B, 32 * 32)

    w1b = conv1_w.reshape(6, 5, 5).astype(bf16)
    s1 = [jnp.einsum('kde,xwe->dxwk', w1b, jnp.asarray(_A1[px], bf16)
                     ).reshape(160, 84) for px in (0, 1)]
    z1 = jnp.zeros((1024, _S1 - _G1), bf16)
    m1 = jnp.concatenate(
        [blk for py in (0, 1) for px in (0, 1) for blk in
         [jnp.pad(s1[px], ((64 * h + 32 * py, 864 - 64 * h - 32 * py), (0, 0)))
          for h in range(14)] + [z1]],
        axis=1)
    b1 = jnp.pad(
        jnp.broadcast_to(conv1_b[None, :], (196, 6)).reshape(1, _G1),
        ((0, 0), (0, _S1 - _G1)))

    w2b = conv2_w.astype(bf16)
    s2 = [jnp.einsum('kcde,xwe->dxcwk', w2b, jnp.asarray(_A2[px], bf16)
                     ).reshape(420, 80) for px in (0, 1)]
    z2 = jnp.zeros((_S1, _S2 - _G2), bf16)
    m2 = jnp.concatenate(
        [blk for py in (0, 1) for px in (0, 1) for blk in
         [jnp.pad(s2[px], (((2 * h + py) * 84,
                            _S1 - 420 - (2 * h + py) * 84), (0, 0)))
          for h in range(5)] + [z2]],
        axis=1)
    b2 = jnp.pad(
        jnp.broadcast_to(conv2_b[None, :], (25, 16)).reshape(1, _G2),
        ((0, 0), (0, _S2 - _G2)))

    w3 = jnp.pad(fc1_w[:, _P2PERM].T.astype(bf16),
                 ((0, _S2 - _G2), (0, 0)))
    w4 = fc2_w.T.astype(bf16)
    w5 = fc3_w.T.astype(bf16)
    b3 = fc1_b.reshape(1, 120)
    b4 = fc2_b.reshape(1, 84)
    b5 = fc3_b.reshape(1, 10)

    pad = (-B) % _BN
    if pad:
        x2d = jnp.pad(x2d, ((0, pad), (0, 0)))
    bp = B + pad

    def const(a):
        return pl.BlockSpec(a.shape, lambda i, _nd=a.ndim: (0,) * _nd)

    out = pl.pallas_call(
        _lenet_block,
        out_shape=jax.ShapeDtypeStruct((bp, 10), jnp.float32),
        grid=(bp // _BN,),
        in_specs=[
            pl.BlockSpec((_BN, 1024), lambda i: (i, 0)),
            const(m1), const(b1), const(m2), const(b2),
            const(w3), const(b3), const(w4), const(b4), const(w5), const(b5),
        ],
        out_specs=pl.BlockSpec((_BN, 10), lambda i: (i, 0)),
        compiler_params=pltpu.CompilerParams(
            dimension_semantics=("parallel",)),
    )(x2d, m1, b1, m2, b2, w3, b3, w4, b4, w5, b5)
    return out[:B] if pad else out


# speedup vs baseline: 2.3833x; 1.2526x over previous
import numpy as np
import jax
import jax.numpy as jnp
from jax.experimental import pallas as pl
from jax.experimental.pallas import tpu as pltpu

_BN = 1024
_S1 = 1280
_S2 = 512
_G1 = 6 * 14 * 14
_G2 = 16 * 5 * 5


def _band(src, half, par):
    a = np.zeros((src, half, 5), np.float32)
    for w in range(half):
        for e in range(5):
            a[2 * w + par + e, w, e] = 1.0
    return a


_A1 = (_band(32, 14, 0), _band(32, 14, 1))
_A2 = (_band(14, 5, 0), _band(14, 5, 1))

_P2PERM = np.arange(400).reshape(16, 5, 5).transpose(1, 2, 0).reshape(400)


def _lenet_block(x_ref, s1a_ref, s1b_ref, b1_ref, s2a_ref, s2b_ref, b2_ref,
                 w3_ref, b3_ref, w4_ref, b4_ref, w5_ref, b5_ref,
                 o_ref, m1_s, m2_s):
    f32 = jnp.float32
    bf16 = jnp.bfloat16

    @pl.when(pl.program_id(1) == 0)
    def _build():
        m1_s[...] = jnp.zeros((1024, 4 * _S1), bf16)
        m2_s[...] = jnp.zeros((_S1, 4 * _S2), bf16)
        s1 = (s1a_ref[...], s1b_ref[...])
        s2 = (s2a_ref[...], s2b_ref[...])
        for py in (0, 1):
            for px in (0, 1):
                g = 2 * py + px
                for h in range(14):
                    r = 64 * h + 32 * py
                    c = g * _S1 + 84 * h
                    m1_s[r:r + 160, c:c + 84] = s1[px]
                for h in range(5):
                    r = (2 * h + py) * 84
                    c = g * _S2 + 80 * h
                    m2_s[r:r + 420, c:c + 80] = s2[px]

    xb = x_ref[...]

    p1 = jnp.dot(xb, m1_s[:, 0:_S1], preferred_element_type=f32)
    for g in range(1, 4):
        p1 = jnp.maximum(p1, jnp.dot(xb, m1_s[:, g * _S1:(g + 1) * _S1],
                                     preferred_element_type=f32))
    p1 = jnp.maximum(p1 + b1_ref[...], 0.0)

    p1b = p1.astype(bf16)
    p2 = jnp.dot(p1b, m2_s[:, 0:_S2], preferred_element_type=f32)
    for g in range(1, 4):
        p2 = jnp.maximum(p2, jnp.dot(p1b, m2_s[:, g * _S2:(g + 1) * _S2],
                                     preferred_element_type=f32))
    p2 = jnp.maximum(p2 + b2_ref[...], 0.0)

    h1 = jnp.maximum(jnp.dot(p2.astype(bf16), w3_ref[...],
                             preferred_element_type=f32) + b3_ref[...], 0.0)
    h2 = jnp.maximum(jnp.dot(h1.astype(bf16), w4_ref[...],
                             preferred_element_type=f32) + b4_ref[...], 0.0)
    o_ref[...] = jnp.dot(h2.astype(bf16), w5_ref[...],
                         preferred_element_type=f32) + b5_ref[...]


@jax.jit
def kernel(x, conv1_w, conv1_b, conv2_w, conv2_b,
           fc1_w, fc1_b, fc2_w, fc2_b, fc3_w, fc3_b):
    bf16 = jnp.bfloat16
    B = x.shape[0]
    x2d = x.reshape(B, 32 * 32).astype(bf16)

    w1b = conv1_w.reshape(6, 5, 5).astype(bf16)
    s1 = [jnp.einsum('kde,xwe->dxwk', w1b, jnp.asarray(_A1[px], bf16)
                     ).reshape(160, 84) for px in (0, 1)]
    b1 = jnp.pad(
        jnp.broadcast_to(conv1_b[None, :], (196, 6)).reshape(1, _G1),
        ((0, 0), (0, _S1 - _G1)))

    w2b = conv2_w.astype(bf16)
    s2 = [jnp.einsum('kcde,xwe->dxcwk', w2b, jnp.asarray(_A2[px], bf16)
                     ).reshape(420, 80) for px in (0, 1)]
    b2 = jnp.pad(
        jnp.broadcast_to(conv2_b[None, :], (25, 16)).reshape(1, _G2),
        ((0, 0), (0, _S2 - _G2)))

    w3 = jnp.pad(fc1_w[:, _P2PERM].T.astype(bf16),
                 ((0, _S2 - _G2), (0, 0)))
    w4 = fc2_w.T.astype(bf16)
    w5 = fc3_w.T.astype(bf16)
    b3 = fc1_b.reshape(1, 120)
    b4 = fc2_b.reshape(1, 84)
    b5 = fc3_b.reshape(1, 10)

    pad = (-B) % (2 * _BN)
    if pad:
        x2d = jnp.pad(x2d, ((0, pad), (0, 0)))
    bp = B + pad
    inner = bp // _BN // 2

    def const(a):
        return pl.BlockSpec(a.shape, lambda i, j, _nd=a.ndim: (0,) * _nd)

    out = pl.pallas_call(
        _lenet_block,
        out_shape=jax.ShapeDtypeStruct((bp, 10), jnp.float32),
        grid=(2, inner),
        in_specs=[
            pl.BlockSpec((_BN, 1024), lambda i, j, _n=inner: (i * _n + j, 0)),
            const(s1[0]), const(s1[1]), const(b1),
            const(s2[0]), const(s2[1]), const(b2),
            const(w3), const(b3), const(w4), const(b4), const(w5), const(b5),
        ],
        out_specs=pl.BlockSpec((_BN, 10),
                               lambda i, j, _n=inner: (i * _n + j, 0)),
        scratch_shapes=[pltpu.VMEM((1024, 4 * _S1), bf16),
                        pltpu.VMEM((_S1, 4 * _S2), bf16)],
        compiler_params=pltpu.CompilerParams(
            dimension_semantics=("parallel", "arbitrary")),
    )(x2d, s1[0], s1[1], b1, s2[0], s2[1], b2,
      w3, b3, w4, b4, w5, b5)
    return out[:B] if pad else out
